# Initial kernel scaffold; baseline (speedup 1.0000x reference)
import functools

import jax
import jax.numpy as jnp
from jax import lax
from jax.experimental import pallas as pl
from jax.experimental.pallas import tpu as pltpu
from jax.experimental.pallas import tpu_sc as plsc

VOCAB = 100000
D = 64
NC = 2
NS = 16
NW = NC * NS
CHUNK = 128


def _dense_body(emb_ref, w1_ref, b1_ref, w2_ref, b2_ref, out_ref):
    x = emb_ref[...]
    h1 = jnp.maximum(
        jnp.dot(x, w1_ref[...], preferred_element_type=jnp.float32) + b1_ref[...],
        0.0,
    )
    out_ref[...] = (
        jnp.dot(h1, w2_ref[...], preferred_element_type=jnp.float32) + b2_ref[...]
    )


def _dense_transform(emb, W1, b1, W2, b2):
    rows_per_block = 2000
    grid = VOCAB // rows_per_block
    return pl.pallas_call(
        _dense_body,
        grid=(grid,),
        in_specs=[
            pl.BlockSpec((rows_per_block, D), lambda i: (i, 0)),
            pl.BlockSpec((D, D), lambda i: (0, 0)),
            pl.BlockSpec((1, D), lambda i: (0, 0)),
            pl.BlockSpec((D, D), lambda i: (0, 0)),
            pl.BlockSpec((1, D), lambda i: (0, 0)),
        ],
        out_specs=pl.BlockSpec((rows_per_block, D), lambda i: (i, 0)),
        out_shape=jax.ShapeDtypeStruct((VOCAB, D), jnp.float32),
    )(emb, W1, b1.reshape(1, D), W2, b2.reshape(1, D))


def _gather_body(h_hbm, idx_hbm, out_hbm, idx_v, rows_a, rows_b, sem_a, sem_b):
    cid = lax.axis_index("c")
    sid = lax.axis_index("s")
    w = sid * NC + cid
    n_chunks = idx_hbm.shape[1]
    pltpu.sync_copy(idx_hbm.at[w], idx_v)
    out_base = w * n_chunks

    def step(i, carry):
        c0 = 2 * i
        c1 = c0 + 1
        cp0 = pltpu.async_copy(h_hbm.at[idx_v.at[c0]], rows_a, sem_a)
        cp1 = pltpu.async_copy(h_hbm.at[idx_v.at[c1]], rows_b, sem_b)
        cp0.wait()
        pltpu.sync_copy(rows_a, out_hbm.at[out_base + c0])
        cp1.wait()
        pltpu.sync_copy(rows_b, out_hbm.at[out_base + c1])
        return carry

    lax.fori_loop(0, n_chunks // 2, step, 0)


def _gather_rows(h, idx3, n_chunks):
    mesh = plsc.VectorSubcoreMesh(
        core_axis_name="c", subcore_axis_name="s", num_cores=NC, num_subcores=NS
    )
    f = pl.kernel(
        _gather_body,
        out_type=jax.ShapeDtypeStruct((NW * n_chunks, CHUNK, D), jnp.float32),
        mesh=mesh,
        scratch_types=[
            pltpu.VMEM((n_chunks, CHUNK), jnp.int32),
            pltpu.VMEM((CHUNK, D), jnp.float32),
            pltpu.VMEM((CHUNK, D), jnp.float32),
            pltpu.SemaphoreType.DMA,
            pltpu.SemaphoreType.DMA,
        ],
    )
    return f(h, idx3)


def kernel(word_ids, emb, W1, b1, W2, b2, edge_index):
    B, S = word_ids.shape
    h = _dense_transform(emb, W1, b1, W2, b2)
    n = B * S
    n_chunks = n // (NW * CHUNK)
    idx3 = word_ids.reshape(NW, n_chunks, CHUNK)
    out = _gather_rows(h, idx3, n_chunks)
    return out.reshape(B, S, D)


# speedup vs baseline: 8.0791x; 8.0791x over previous
import functools

import jax
import jax.numpy as jnp
from jax import lax
from jax.experimental import pallas as pl
from jax.experimental.pallas import tpu as pltpu
from jax.experimental.pallas import tpu_sc as plsc

VOCAB = 100000
D = 64
NC = 2
NS = 16
NW = NC * NS
CHUNK = 128


def _dense_body(emb_ref, w1_ref, b1_ref, w2_ref, b2_ref, out_ref):
    x = emb_ref[...]
    h1 = jnp.maximum(
        jnp.dot(x, w1_ref[...], preferred_element_type=jnp.float32) + b1_ref[...],
        0.0,
    )
    out_ref[...] = (
        jnp.dot(h1, w2_ref[...], preferred_element_type=jnp.float32) + b2_ref[...]
    )


def _dense_transform(emb, W1, b1, W2, b2):
    rows_per_block = 2000
    grid = VOCAB // rows_per_block
    return pl.pallas_call(
        _dense_body,
        grid=(grid,),
        in_specs=[
            pl.BlockSpec((rows_per_block, D), lambda i: (i, 0)),
            pl.BlockSpec((D, D), lambda i: (0, 0)),
            pl.BlockSpec((1, D), lambda i: (0, 0)),
            pl.BlockSpec((D, D), lambda i: (0, 0)),
            pl.BlockSpec((1, D), lambda i: (0, 0)),
        ],
        out_specs=pl.BlockSpec((rows_per_block, D), lambda i: (i, 0)),
        out_shape=jax.ShapeDtypeStruct((VOCAB, D), jnp.float32),
    )(emb, W1, b1.reshape(1, D), W2, b2.reshape(1, D))


def _gather_body(h_hbm, idx_hbm, out_hbm, idx_v, rows_a, rows_b, sem_a, sem_b):
    cid = lax.axis_index("c")
    sid = lax.axis_index("s")
    w = sid * NC + cid
    n_chunks = idx_hbm.shape[1]
    pltpu.sync_copy(idx_hbm.at[w], idx_v)
    out_base = w * n_chunks

    def step(i, carry):
        c0 = 2 * i
        c1 = c0 + 1
        cp0 = pltpu.async_copy(h_hbm.at[idx_v.at[c0]], rows_a, sem_a)
        cp1 = pltpu.async_copy(h_hbm.at[idx_v.at[c1]], rows_b, sem_b)
        cp0.wait()
        pltpu.sync_copy(rows_a, out_hbm.at[out_base + c0])
        cp1.wait()
        pltpu.sync_copy(rows_b, out_hbm.at[out_base + c1])
        return carry

    lax.fori_loop(0, n_chunks // 2, step, 0)


def _gather_rows(h, idx3, n_chunks):
    mesh = plsc.VectorSubcoreMesh(
        core_axis_name="c", subcore_axis_name="s", num_cores=NC, num_subcores=NS
    )
    f = pl.kernel(
        _gather_body,
        out_type=jax.ShapeDtypeStruct((NW * n_chunks, CHUNK, D), jnp.float32),
        mesh=mesh,
        scratch_types=[
            pltpu.VMEM((n_chunks, CHUNK), jnp.int32),
            pltpu.VMEM((CHUNK, D), jnp.float32),
            pltpu.VMEM((CHUNK, D), jnp.float32),
            pltpu.SemaphoreType.DMA,
            pltpu.SemaphoreType.DMA,
        ],
        compiler_params=pltpu.CompilerParams(use_tc_tiling_on_sc=False),
    )
    return f(h, idx3)


def kernel(word_ids, emb, W1, b1, W2, b2, edge_index):
    B, S = word_ids.shape
    h = _dense_transform(emb, W1, b1, W2, b2)
    n = B * S
    n_chunks = n // (NW * CHUNK)
    idx3 = word_ids.reshape(NW, n_chunks, CHUNK)
    out = _gather_rows(h, idx3, n_chunks)
    return out.reshape(B, S, D)


# speedup vs baseline: 8.6894x; 1.0755x over previous
import functools

import jax
import jax.numpy as jnp
from jax import lax
from jax.experimental import pallas as pl
from jax.experimental.pallas import tpu as pltpu
from jax.experimental.pallas import tpu_sc as plsc

VOCAB = 100000
D = 64
NC = 2
NS = 16
NW = NC * NS
CHUNK = 128


def _dense_body(emb_ref, w1_ref, b1_ref, w2_ref, b2_ref, out_ref):
    x = emb_ref[...]
    h1 = jnp.maximum(
        jnp.dot(x, w1_ref[...], preferred_element_type=jnp.float32) + b1_ref[...],
        0.0,
    )
    out_ref[...] = (
        jnp.dot(h1, w2_ref[...], preferred_element_type=jnp.float32) + b2_ref[...]
    )


def _dense_transform(emb, W1, b1, W2, b2):
    rows_per_block = 2000
    grid = VOCAB // rows_per_block
    return pl.pallas_call(
        _dense_body,
        grid=(grid,),
        in_specs=[
            pl.BlockSpec((rows_per_block, D), lambda i: (i, 0)),
            pl.BlockSpec((D, D), lambda i: (0, 0)),
            pl.BlockSpec((1, D), lambda i: (0, 0)),
            pl.BlockSpec((D, D), lambda i: (0, 0)),
            pl.BlockSpec((1, D), lambda i: (0, 0)),
        ],
        out_specs=pl.BlockSpec((rows_per_block, D), lambda i: (i, 0)),
        out_shape=jax.ShapeDtypeStruct((VOCAB, D), jnp.float32),
    )(emb, W1, b1.reshape(1, D), W2, b2.reshape(1, D))


GPC = 4
GROUP = GPC * CHUNK


def _gather_body(h_hbm, idx_hbm, out_hbm, idx_v, rows_a, rows_b, gsem, wsem):
    cid = lax.axis_index("c")
    sid = lax.axis_index("s")
    w = sid * NC + cid
    n_chunks = idx_hbm.shape[1]
    n_groups = n_chunks // GPC
    pltpu.sync_copy(idx_hbm.at[w], idx_v)
    out_base = w * n_groups
    bufs = (rows_a, rows_b)

    def step(i, carry):
        descs = []
        for half in range(2):
            g = 2 * i + half
            buf = bufs[half]

            @pl.when(i > 0)
            def _(buf=buf):
                pltpu.make_async_copy(buf, out_hbm.at[out_base], wsem).wait()

            d = []
            for j in range(GPC):
                d.append(
                    pltpu.async_copy(
                        h_hbm.at[idx_v.at[g * GPC + j]],
                        buf.at[pl.ds(j * CHUNK, CHUNK)],
                        gsem,
                    )
                )
            descs.append(d)
        for half in range(2):
            g = 2 * i + half
            buf = bufs[half]
            for d in descs[half]:
                d.wait()
            pltpu.async_copy(buf, out_hbm.at[out_base + g], wsem)
        return carry

    lax.fori_loop(0, n_groups // 2, step, 0)
    pltpu.make_async_copy(rows_a, out_hbm.at[out_base], wsem).wait()
    pltpu.make_async_copy(rows_b, out_hbm.at[out_base], wsem).wait()


def _gather_rows(h, idx3, n_chunks):
    n_groups = n_chunks // GPC
    mesh = plsc.VectorSubcoreMesh(
        core_axis_name="c", subcore_axis_name="s", num_cores=NC, num_subcores=NS
    )
    f = pl.kernel(
        _gather_body,
        out_type=jax.ShapeDtypeStruct((NW * n_groups, GROUP, D), jnp.float32),
        mesh=mesh,
        scratch_types=[
            pltpu.VMEM((n_chunks, CHUNK), jnp.int32),
            pltpu.VMEM((GROUP, D), jnp.float32),
            pltpu.VMEM((GROUP, D), jnp.float32),
            pltpu.SemaphoreType.DMA,
            pltpu.SemaphoreType.DMA,
        ],
        compiler_params=pltpu.CompilerParams(use_tc_tiling_on_sc=False),
    )
    return f(h, idx3)


def kernel(word_ids, emb, W1, b1, W2, b2, edge_index):
    B, S = word_ids.shape
    h = _dense_transform(emb, W1, b1, W2, b2)
    n = B * S
    n_chunks = n // (NW * CHUNK)
    idx3 = word_ids.reshape(NW, n_chunks, CHUNK)
    out = _gather_rows(h, idx3, n_chunks)
    return out.reshape(B, S, D)
